# baseline (speedup 1.0000x reference)
import functools

import jax
import jax.numpy as jnp
from jax import lax
from jax.experimental import pallas as pl
from jax.experimental.pallas import tpu as pltpu
from jax.experimental.pallas import tpu_sc as plsc

_V = 100000
_D = 64
_B = 1024
_L = 50

_NC = 2
_NS = 16
_NW = _NC * _NS
_BPW = _B // _NW
_RPW = _BPW * _L
_CHUNK = 128

_mesh = plsc.VectorSubcoreMesh(core_axis_name="c", subcore_axis_name="s")


@functools.partial(
    pl.kernel,
    mesh=_mesh,
    out_type=jax.ShapeDtypeStruct((_B, _D), jnp.float32),
    scratch_types=[
        pltpu.VMEM((_RPW,), jnp.int32),
        pltpu.VMEM((_RPW, _D), jnp.float32),
        pltpu.VMEM((_RPW,), jnp.float32),
        pltpu.VMEM((_BPW, _D), jnp.float32),
        pltpu.SemaphoreType.DMA,
    ],
    compiler_params=pltpu.CompilerParams(
        needs_layout_passes=False, use_tc_tiling_on_sc=False),
)
def _sc_pool(idx_hbm, emb_hbm, x_hbm, idx_v, rows_v, scales_v, x_v, sem):
    wid = lax.axis_index("s") * _NC + lax.axis_index("c")
    base = wid * _RPW
    pltpu.sync_copy(idx_hbm.at[pl.ds(base, _RPW)], idx_v)

    copies = []
    off = 0
    while off < _RPW:
        sz = min(_CHUNK, _RPW - off)
        copies.append(pltpu.async_copy(
            emb_hbm.at[idx_v.at[pl.ds(off, sz)]],
            rows_v.at[pl.ds(off, sz)],
            sem,
        ))
        off += sz
    for cp in copies:
        cp.wait()

    def g_body(g, carry):
        r0 = g * 16
        row_ids = r0 + lax.iota(jnp.int32, 16)
        acc = jnp.zeros((16,), jnp.float32)
        for d in range(_D):
            col = jnp.full((16,), d, jnp.int32)
            v = plsc.load_gather(rows_v, [row_ids, col])
            acc = acc + v * v
        i = plsc.bitcast(acc, jnp.int32)
        y = plsc.bitcast(jnp.int32(0x5F3759DF) - (i >> 1), jnp.float32)
        y = y * (1.5 - 0.5 * acc * y * y)
        y = y * (1.5 - 0.5 * acc * y * y)
        scales_v[pl.ds(r0, 16)] = jnp.where(
            acc > 1.0, y, jnp.float32(1.0)) * jnp.float32(1.0 / _L)
        return carry

    lax.fori_loop(0, _RPW // 16, g_body, 0)

    def b_body(bb, carry):
        def l_body(l, accs):
            a0, a1, a2, a3 = accs
            r = bb * _L + l
            s = plsc.load_gather(scales_v, [jnp.full((16,), r, jnp.int32)])
            return (a0 + rows_v[r, pl.ds(0, 16)] * s,
                    a1 + rows_v[r, pl.ds(16, 16)] * s,
                    a2 + rows_v[r, pl.ds(32, 16)] * s,
                    a3 + rows_v[r, pl.ds(48, 16)] * s)

        z = jnp.zeros((16,), jnp.float32)
        a0, a1, a2, a3 = lax.fori_loop(0, _L, l_body, (z, z, z, z))
        x_v[bb, pl.ds(0, 16)] = a0
        x_v[bb, pl.ds(16, 16)] = a1
        x_v[bb, pl.ds(32, 16)] = a2
        x_v[bb, pl.ds(48, 16)] = a3
        return carry

    lax.fori_loop(0, _BPW, b_body, 0)
    pltpu.sync_copy(x_v, x_hbm.at[pl.ds(wid * _BPW, _BPW)])


_VT = 512


def _mm_body(x_ref, w_ref, b_ref, o_ref):
    o_ref[...] = lax.dot_general(
        x_ref[...], w_ref[...],
        dimension_numbers=(((1,), (1,)), ((), ())),
        preferred_element_type=jnp.float32,
    ) + b_ref[...]


def _project(x, W, b2):
    nv = pl.cdiv(_V, _VT)
    return pl.pallas_call(
        _mm_body,
        grid=(nv,),
        in_specs=[
            pl.BlockSpec((_B, _D), lambda j: (0, 0)),
            pl.BlockSpec((_VT, _D), lambda j: (j, 0)),
            pl.BlockSpec((1, _VT), lambda j: (0, j)),
        ],
        out_specs=pl.BlockSpec((_B, _VT), lambda j: (0, j)),
        out_shape=jax.ShapeDtypeStruct((_B, _V), jnp.float32),
        compiler_params=pltpu.CompilerParams(
            dimension_semantics=("parallel",),
        ),
    )(x, W, b2)


def kernel(inputs_, emb, W, b):
    idx = inputs_.reshape(-1).astype(jnp.int32)
    x = _sc_pool(idx, emb)
    return _project(x, W, b.reshape(1, _V))


# speedup vs baseline: 2.6786x; 2.6786x over previous
import functools

import jax
import jax.numpy as jnp
from jax import lax
from jax.experimental import pallas as pl
from jax.experimental.pallas import tpu as pltpu
from jax.experimental.pallas import tpu_sc as plsc

_V = 100000
_D = 64
_B = 1024
_L = 50

_NC = 2
_NS = 16
_NW = _NC * _NS
_BPW = _B // _NW
_RPW = _BPW * _L
_CHUNK = 128

_mesh = plsc.VectorSubcoreMesh(core_axis_name="c", subcore_axis_name="s")


@functools.partial(
    pl.kernel,
    mesh=_mesh,
    out_type=jax.ShapeDtypeStruct((_B, _D), jnp.float32),
    scratch_types=[
        pltpu.VMEM((_RPW,), jnp.int32),
        pltpu.VMEM((_RPW, _D), jnp.float32),
        pltpu.VMEM((_RPW,), jnp.float32),
        pltpu.VMEM((_BPW, _D), jnp.float32),
        pltpu.SemaphoreType.DMA,
    ],
    compiler_params=pltpu.CompilerParams(
        needs_layout_passes=False, use_tc_tiling_on_sc=False),
)
def _sc_pool(idx_hbm, emb_hbm, x_hbm, idx_v, rows_v, scales_v, x_v, sem):
    wid = lax.axis_index("s") * _NC + lax.axis_index("c")
    base = wid * _RPW
    pltpu.sync_copy(idx_hbm.at[pl.ds(base, _RPW)], idx_v)

    copies = []
    off = 0
    while off < _RPW:
        sz = min(_CHUNK, _RPW - off)
        copies.append(pltpu.async_copy(
            emb_hbm.at[idx_v.at[pl.ds(off, sz)]],
            rows_v.at[pl.ds(off, sz)],
            sem,
        ))
        off += sz
    for cp in copies:
        cp.wait()

    def g_body(g, carry):
        r0 = g * 16
        row_ids = r0 + lax.iota(jnp.int32, 16)
        acc = jnp.zeros((16,), jnp.float32)
        for d in range(_D):
            col = jnp.full((16,), d, jnp.int32)
            v = plsc.load_gather(rows_v, [row_ids, col])
            acc = acc + v * v
        i = plsc.bitcast(acc, jnp.int32)
        y = plsc.bitcast(jnp.int32(0x5F3759DF) - (i >> 1), jnp.float32)
        y = y * (1.5 - 0.5 * acc * y * y)
        y = y * (1.5 - 0.5 * acc * y * y)
        scales_v[pl.ds(r0, 16)] = jnp.where(
            acc > 1.0, y, jnp.float32(1.0)) * jnp.float32(1.0 / _L)
        return carry

    lax.fori_loop(0, _RPW // 16, g_body, 0)

    def b_body(bb, carry):
        def l_body(l, accs):
            a0, a1, a2, a3 = accs
            r = bb * _L + l
            s = plsc.load_gather(scales_v, [jnp.full((16,), r, jnp.int32)])
            return (a0 + rows_v[r, pl.ds(0, 16)] * s,
                    a1 + rows_v[r, pl.ds(16, 16)] * s,
                    a2 + rows_v[r, pl.ds(32, 16)] * s,
                    a3 + rows_v[r, pl.ds(48, 16)] * s)

        z = jnp.zeros((16,), jnp.float32)
        a0, a1, a2, a3 = lax.fori_loop(0, _L, l_body, (z, z, z, z))
        x_v[bb, pl.ds(0, 16)] = a0
        x_v[bb, pl.ds(16, 16)] = a1
        x_v[bb, pl.ds(32, 16)] = a2
        x_v[bb, pl.ds(48, 16)] = a3
        return carry

    lax.fori_loop(0, _BPW, b_body, 0)
    pltpu.sync_copy(x_v, x_hbm.at[pl.ds(wid * _BPW, _BPW)])


_VT = 512


def _mm_body(x_ref, w_ref, b_ref, o_ref):
    o_ref[...] = lax.dot_general(
        x_ref[...], w_ref[...],
        dimension_numbers=(((1,), (1,)), ((), ())),
        preferred_element_type=jnp.float32,
    ) + b_ref[...]


def _project(x, W, b2):
    nv = pl.cdiv(_V, _VT)
    return pl.pallas_call(
        _mm_body,
        grid=(nv,),
        in_specs=[
            pl.BlockSpec((_B, _D), lambda j: (0, 0)),
            pl.BlockSpec((_VT, _D), lambda j: (j, 0)),
            pl.BlockSpec((1, _VT), lambda j: (0, j)),
        ],
        out_specs=pl.BlockSpec((_B, _VT), lambda j: (0, j)),
        out_shape=jax.ShapeDtypeStruct((_B, _V), jnp.float32),
        compiler_params=pltpu.CompilerParams(
            dimension_semantics=("parallel",),
        ),
    )(x, W, b2)


def kernel(inputs_, emb, W, b):
    idx = inputs_.reshape(-1).astype(jnp.int32)
    x = _sc_pool(idx, emb)
    return x @ W.T + b
